# Initial kernel scaffold; baseline (speedup 1.0000x reference)
import jax
import jax.numpy as jnp
from jax.experimental import pallas as pl


def kernel(x, router, shared_gate, shared_up, shared_down, gate, up, down):
    raise NotImplementedError("write your pallas kernel here")


# speedup vs baseline: 2.6213x; 2.6213x over previous
import jax
import jax.numpy as jnp
from jax.experimental import pallas as pl
from jax.experimental.pallas import tpu as pltpu

_NE, _K, _D, _H, _N = 8, 2, 1024, 512, 2048


def _moe_dense_body(x_ref, router_ref, sg_ref, su_ref, sd_ref,
                    g_ref, u_ref, d_ref, out_ref, w_ref):
    e = pl.program_id(0)
    x = x_ref[...]

    @pl.when(e == 0)
    def _init():
        logits = jnp.dot(x, router_ref[...], preferred_element_type=jnp.float32)
        m = jnp.max(logits, axis=-1, keepdims=True)
        p = jnp.exp(logits - m)
        p = p / jnp.sum(p, axis=-1, keepdims=True)
        idx = jax.lax.broadcasted_iota(jnp.int32, p.shape, 1)
        m1 = jnp.max(p, axis=-1, keepdims=True)
        i1 = jnp.min(jnp.where(p == m1, idx, _NE), axis=-1, keepdims=True)
        p2 = jnp.where(idx == i1, -jnp.inf, p)
        m2 = jnp.max(p2, axis=-1, keepdims=True)
        i2 = jnp.min(jnp.where(p2 == m2, idx, _NE), axis=-1, keepdims=True)
        w_ref[...] = jnp.where((idx == i1) | (idx == i2), p, 0.0)

        hg = jnp.dot(x, sg_ref[...], preferred_element_type=jnp.float32)
        hu = jnp.dot(x, su_ref[...], preferred_element_type=jnp.float32)
        h = hg * jax.nn.sigmoid(hg) * hu
        out_ref[...] = jnp.dot(h, sd_ref[...], preferred_element_type=jnp.float32)

    hg = jnp.dot(x, g_ref[0], preferred_element_type=jnp.float32)
    hu = jnp.dot(x, u_ref[0], preferred_element_type=jnp.float32)
    wf = w_ref[...]
    col = jax.lax.broadcasted_iota(jnp.int32, wf.shape, 1)
    w = jnp.sum(jnp.where(col == e, wf, 0.0), axis=1, keepdims=True)
    h = hg * jax.nn.sigmoid(hg) * hu * w
    out_ref[...] += jnp.dot(h, d_ref[0], preferred_element_type=jnp.float32)


def kernel(x, router, shared_gate, shared_up, shared_down, gate, up, down):
    Bx, Tx, D = x.shape
    x_flat = x.reshape(Bx * Tx, D)
    out = pl.pallas_call(
        _moe_dense_body,
        grid=(_NE,),
        in_specs=[
            pl.BlockSpec((_N, _D), lambda e: (0, 0)),
            pl.BlockSpec((_D, _NE), lambda e: (0, 0)),
            pl.BlockSpec((_D, _H), lambda e: (0, 0)),
            pl.BlockSpec((_D, _H), lambda e: (0, 0)),
            pl.BlockSpec((_H, _D), lambda e: (0, 0)),
            pl.BlockSpec((1, _D, _H), lambda e: (e, 0, 0)),
            pl.BlockSpec((1, _D, _H), lambda e: (e, 0, 0)),
            pl.BlockSpec((1, _H, _D), lambda e: (e, 0, 0)),
        ],
        out_specs=pl.BlockSpec((_N, _D), lambda e: (0, 0)),
        out_shape=jax.ShapeDtypeStruct((_N, _D), jnp.float32),
        scratch_shapes=[pltpu.VMEM((_N, _NE), jnp.float32)],
    )(x_flat, router, shared_gate, shared_up, shared_down, gate, up, down)
    return out.reshape(Bx, Tx, D)
